# Initial kernel scaffold; baseline (speedup 1.0000x reference)
import jax
import jax.numpy as jnp
from jax.experimental import pallas as pl


def kernel(data, w1):
    raise NotImplementedError("write your pallas kernel here")


# speedup vs baseline: 4.6524x; 4.6524x over previous
import functools

import jax
import jax.numpy as jnp
from jax.experimental import pallas as pl

_PROXIES = 16
_EMBED = 512


def _fused_kernel(d_ref, w_ref, o_ref, *, bm, bn):
    d = d_ref[...]
    norm = jnp.sqrt(jnp.sum(d * d, axis=1, keepdims=True))
    dn = d * (1.0 / jnp.maximum(norm, 1e-12))
    x = jax.lax.dot_general(
        w_ref[...], dn,
        (((1,), (1,)), ((), ())),
        preferred_element_type=jnp.float32,
    )
    o_ref[...] = jnp.max(x.reshape(bm // _PROXIES, _PROXIES, bn), axis=1)


def kernel(data, w1):
    batch, embed = data.shape
    rows = w1.shape[0]
    num_classes = rows // _PROXIES

    bm = 1024
    bn = batch
    grid = (rows // bm,)

    fn = functools.partial(_fused_kernel, bm=bm, bn=bn)
    out = pl.pallas_call(
        fn,
        grid=grid,
        in_specs=[
            pl.BlockSpec((bn, embed), lambda i: (0, 0)),
            pl.BlockSpec((bm, embed), lambda i: (i, 0)),
        ],
        out_specs=pl.BlockSpec((bm // _PROXIES, bn), lambda i: (i, 0)),
        out_shape=jax.ShapeDtypeStruct((num_classes, batch), jnp.float32),
    )(data, w1)
    return out


# speedup vs baseline: 4.8044x; 1.0327x over previous
import functools

import jax
import jax.numpy as jnp
from jax.experimental import pallas as pl
from jax.experimental.pallas import tpu as pltpu

_PROXIES = 16


def _fused_kernel(d_ref, w_ref, o_ref, db_ref, *, bm, bn):
    @pl.when(pl.program_id(0) == 0)
    def _():
        d = d_ref[...]
        norm = jnp.sqrt(jnp.sum(d * d, axis=1, keepdims=True))
        db_ref[...] = (d * (1.0 / jnp.maximum(norm, 1e-12))).astype(jnp.bfloat16)

    x = jax.lax.dot_general(
        w_ref[...].astype(jnp.bfloat16), db_ref[...],
        (((1,), (1,)), ((), ())),
        preferred_element_type=jnp.float32,
    )
    o_ref[...] = jnp.max(x.reshape(bm // _PROXIES, _PROXIES, bn), axis=1)


def kernel(data, w1):
    batch, embed = data.shape
    rows = w1.shape[0]
    num_classes = rows // _PROXIES

    bm = 1024
    bn = batch
    grid = (rows // bm,)

    fn = functools.partial(_fused_kernel, bm=bm, bn=bn)
    out = pl.pallas_call(
        fn,
        grid=grid,
        in_specs=[
            pl.BlockSpec((bn, embed), lambda i: (0, 0)),
            pl.BlockSpec((bm, embed), lambda i: (i, 0)),
        ],
        out_specs=pl.BlockSpec((bm // _PROXIES, bn), lambda i: (i, 0)),
        out_shape=jax.ShapeDtypeStruct((num_classes, batch), jnp.float32),
        scratch_shapes=[pltpu.VMEM((bn, embed), jnp.bfloat16)],
    )(data, w1)
    return out


# speedup vs baseline: 4.8251x; 1.0043x over previous
import functools

import jax
import jax.numpy as jnp
from jax.experimental import pallas as pl
from jax.experimental.pallas import tpu as pltpu

_PROXIES = 16


def _fused_kernel(d_ref, w_ref, o_ref, db_ref, *, bm, bn):
    @pl.when(pl.program_id(0) == 0)
    def _():
        d = d_ref[...]
        norm = jnp.sqrt(jnp.sum(d * d, axis=1, keepdims=True))
        db_ref[...] = (d * (1.0 / jnp.maximum(norm, 1e-12))).astype(jnp.bfloat16)

    x = jax.lax.dot_general(
        w_ref[...].astype(jnp.bfloat16), db_ref[...],
        (((1,), (1,)), ((), ())),
        preferred_element_type=jnp.float32,
    )
    o_ref[...] = jnp.max(x.reshape(bm // _PROXIES, _PROXIES, bn), axis=1)


def kernel(data, w1):
    batch, embed = data.shape
    rows = w1.shape[0]
    num_classes = rows // _PROXIES

    bm = 2048
    bn = batch
    grid = (rows // bm,)

    fn = functools.partial(_fused_kernel, bm=bm, bn=bn)
    out = pl.pallas_call(
        fn,
        grid=grid,
        in_specs=[
            pl.BlockSpec((bn, embed), lambda i: (0, 0)),
            pl.BlockSpec((bm, embed), lambda i: (i, 0)),
        ],
        out_specs=pl.BlockSpec((bm // _PROXIES, bn), lambda i: (i, 0)),
        out_shape=jax.ShapeDtypeStruct((num_classes, batch), jnp.float32),
        scratch_shapes=[pltpu.VMEM((bn, embed), jnp.bfloat16)],
    )(data, w1)
    return out
